# baseline (speedup 1.0000x reference)
import functools

import jax
import jax.numpy as jnp
from jax import lax
from jax.experimental import pallas as pl
from jax.experimental.pallas import tpu as pltpu
from jax.experimental.pallas import tpu_sc as plsc

B = 16384
F = 26
FPAD = 32
V = 1000000
D = 16

NC = 2
NS = 16
NW = NC * NS
RW = B // NW
CB = 128
NCHUNK = RW // CB
FP = 512
NT = FP // 128
GPC = CB * 8


def _sc_gather_body(emb_hbm, bias_hbm, feat_hbm, fv_hbm,
                    rows_out, first_out,
                    feat_blk, fv_blk, idx_v, stage_v, bias_v, acc_v,
                    sem_r, sem_b):
    wid = lax.axis_index("s") * NC + lax.axis_index("c")
    b0 = wid * RW
    lanes = lax.iota(jnp.int32, 16)

    pltpu.sync_copy(feat_hbm.at[pl.ds(b0, RW), :], feat_blk)
    pltpu.sync_copy(fv_hbm.at[pl.ds(b0, RW), :], fv_blk)

    def zero(k, carry):
        acc_v[pl.ds(k * 16, 16)] = jnp.zeros((16,), jnp.float32)
        return carry

    lax.fori_loop(0, RW // 16, zero, 0)

    def chunk_body(ct, carry):
        c = ct // NT
        t = ct % NT
        bc = c * CB

        def build(g, carry):
            i = g * 16 + lanes
            fl = i // CB
            bl = i % CB
            f = jnp.minimum(t * 8 + fl, F - 1)
            idx_v[pl.ds(g * 16, 16)] = plsc.bitcast(
                plsc.load_gather(feat_blk, [bc + bl, f]), jnp.int32)
            return carry

        lax.fori_loop(0, GPC // 16, build, 0)
        pltpu.async_copy(emb_hbm.at[idx_v], stage_v, sem_r).wait()

        def wr(fl, carry):
            pltpu.sync_copy(
                stage_v.at[pl.ds(fl * CB, CB), :],
                rows_out.at[t, pl.ds(b0 + bc, CB), pl.ds(fl * D, D)])
            return carry

        lax.fori_loop(0, 8, wr, 0)
        return carry

    lax.fori_loop(0, NCHUNK * NT, chunk_body, 0)

    def fo_body(f, carry):
        fcol = jnp.full((16,), f, jnp.int32)

        def pick(k, carry):
            idx_v[pl.ds(k * 16, 16)] = plsc.bitcast(
                plsc.load_gather(feat_blk, [k * 16 + lanes, fcol]),
                jnp.int32)
            return carry

        lax.fori_loop(0, RW // 16, pick, 0)
        pltpu.async_copy(bias_hbm.at[idx_v.at[pl.ds(0, RW)]], bias_v,
                         sem_b).wait()

        def fma(k, carry):
            s = pl.ds(k * 16, 16)
            fv_vec = plsc.load_gather(fv_blk, [k * 16 + lanes, fcol])
            acc_v[s] = acc_v[s] + bias_v[s] * fv_vec
            return carry

        lax.fori_loop(0, RW // 16, fma, 0)
        return carry

    lax.fori_loop(0, F, fo_body, 0)
    pltpu.sync_copy(acc_v, first_out.at[pl.ds(b0, RW)])


_sc_gather = functools.partial(
    pl.kernel,
    out_type=[
        jax.ShapeDtypeStruct((NT, B, 128), jnp.float32),
        jax.ShapeDtypeStruct((B,), jnp.float32),
    ],
    mesh=plsc.VectorSubcoreMesh(core_axis_name="c", subcore_axis_name="s"),
    scratch_types=[
        pltpu.VMEM((RW, FPAD), jnp.float32),
        pltpu.VMEM((RW, FPAD), jnp.float32),
        pltpu.VMEM((GPC,), jnp.int32),
        pltpu.VMEM((GPC, D), jnp.float32),
        pltpu.VMEM((RW,), jnp.float32),
        pltpu.VMEM((RW,), jnp.float32),
        pltpu.SemaphoreType.DMA,
        pltpu.SemaphoreType.DMA,
    ],
    compiler_params=pltpu.CompilerParams(use_tc_tiling_on_sc=False,
                                         needs_layout_passes=False),
)(_sc_gather_body)


BM = 256


def _tc_body(scal_ref, rows_ref, fv_ref, e_ref, p_ref,
             w1t_ref, b1_ref, w2t_ref, b2_ref, wph_ref, out_ref):
    fv = fv_ref[...]
    acc_s1 = jnp.zeros((BM, D), jnp.float32)
    acc_sq = jnp.zeros((BM, D), jnp.float32)
    acc_h = jnp.zeros((BM, 256), jnp.float32)
    for t in range(NT):
        ts = pl.ds(t * 128, 128)
        xt = rows_ref[t]
        fvet = jnp.dot(fv, e_ref[:, ts],
                       preferred_element_type=jnp.float32)
        fet = xt * fvet
        acc_s1 = acc_s1 + jnp.dot(fet, p_ref[ts, :],
                                  preferred_element_type=jnp.float32)
        acc_sq = acc_sq + jnp.dot(fet * fet, p_ref[ts, :],
                                  preferred_element_type=jnp.float32)
        acc_h = acc_h + jnp.dot(fet, w1t_ref[ts, :],
                                preferred_element_type=jnp.float32)
    second = 0.5 * jnp.sum(acc_s1 * acc_s1 - acc_sq, axis=1, keepdims=True)
    h = jnp.maximum(acc_h + b1_ref[...], 0.0)
    h = jnp.dot(h, w2t_ref[...], preferred_element_type=jnp.float32)
    h = jnp.maximum(h + b2_ref[...], 0.0)
    o = jnp.dot(h, wph_ref[...], preferred_element_type=jnp.float32)
    out_ref[...] = o + second * scal_ref[1] + scal_ref[2]


def _tc_fused(rows3, fv, e_mat, p_mat, w1t, b1r, w2t, b2r, wph, scal):
    grid = (B // BM,)
    full2 = lambda shape: pl.BlockSpec(shape, lambda i: (0, 0))
    return pl.pallas_call(
        _tc_body,
        grid=grid,
        in_specs=[
            pl.BlockSpec(memory_space=pltpu.SMEM),
            pl.BlockSpec((NT, BM, 128), lambda i: (0, i, 0)),
            pl.BlockSpec((BM, F), lambda i: (i, 0)),
            full2((F, FP)),
            full2((FP, D)),
            full2((FP, 256)),
            full2((1, 256)),
            full2((256, 128)),
            full2((1, 128)),
            full2((128, 1)),
        ],
        out_specs=pl.BlockSpec((BM, 1), lambda i: (i, 0)),
        out_shape=jax.ShapeDtypeStruct((B, 1), jnp.float32),
    )(scal, rows3, fv, e_mat, p_mat, w1t, b1r, w2t, b2r, wph)


def kernel(features, feature_values, emb_table, bias_table,
           W1, b1, W2, b2, Wp, bp):
    bias_flat = bias_table.reshape(-1)
    feat_p = jnp.pad(
        lax.bitcast_convert_type(features.astype(jnp.int32), jnp.float32),
        ((0, 0), (0, FPAD - F)))
    fv_p = jnp.pad(feature_values, ((0, 0), (0, FPAD - F)))

    rows3, first = _sc_gather(emb_table, bias_flat, feat_p, fv_p)

    cols = jnp.arange(FP, dtype=jnp.int32)
    fcol = (cols // 128) * 8 + (cols % 128) // D
    dcol = cols % D
    valid = fcol < F
    e_mat = ((fcol[None, :] == jnp.arange(F, dtype=jnp.int32)[:, None])
             & valid[None, :]).astype(jnp.float32)
    p_mat = ((dcol[:, None] == jnp.arange(D, dtype=jnp.int32)[None, :])
             & valid[:, None]).astype(jnp.float32)
    w1t_pad = jnp.pad(W1.T, ((0, FP - F * D), (0, 0)))
    scal = jnp.concatenate([Wp[0, :2], bp]).astype(jnp.float32)

    out = _tc_fused(rows3, feature_values, e_mat, p_mat,
                    w1t_pad, b1.reshape(1, -1), W2.T, b2.reshape(1, -1),
                    Wp[0, 2:].reshape(-1, 1), scal)
    return out.reshape(-1) + first * Wp[0, 0]


# speedup vs baseline: 1.1165x; 1.1165x over previous
import functools

import jax
import jax.numpy as jnp
from jax import lax
from jax.experimental import pallas as pl
from jax.experimental.pallas import tpu as pltpu
from jax.experimental.pallas import tpu_sc as plsc

B = 16384
F = 26
FPAD = 32
V = 1000000
D = 16

NC = 2
NS = 16
NW = NC * NS
RW = B // NW
CB = 128
NCHUNK = RW // CB
FP = 512
NT = FP // 128
GPC = CB * 8


NCT = NCHUNK * NT
IDXW = NCT * GPC


def _sc_prep_body(bias_hbm, feat_hbm, fv_hbm,
                  idx_out, first_out,
                  feat_blk, fv_blk, idx_v, bias_v, acc_v, sem_b):
    wid = lax.axis_index("s") * NC + lax.axis_index("c")
    b0 = wid * RW
    lanes = lax.iota(jnp.int32, 16)

    pltpu.sync_copy(feat_hbm.at[pl.ds(b0, RW), :], feat_blk)
    pltpu.sync_copy(fv_hbm.at[pl.ds(b0, RW), :], fv_blk)

    def zero(k, carry):
        acc_v[pl.ds(k * 16, 16)] = jnp.zeros((16,), jnp.float32)
        return carry

    lax.fori_loop(0, RW // 16, zero, 0)

    def chunk_body(ct, carry):
        c = ct // NT
        t = ct % NT
        bc = c * CB

        def build(g, carry):
            i = g * 16 + lanes
            fl = i // CB
            bl = i % CB
            f = jnp.minimum(t * 8 + fl, F - 1)
            idx_v[pl.ds(g * 16, 16)] = plsc.bitcast(
                plsc.load_gather(feat_blk, [bc + bl, f]), jnp.int32)
            return carry

        lax.fori_loop(0, GPC // 16, build, 0)
        pltpu.sync_copy(idx_v, idx_out.at[pl.ds(wid * IDXW + ct * GPC, GPC)])
        cp_b = pltpu.async_copy(bias_hbm.at[idx_v], bias_v, sem_b)
        cp_b.wait()

        def fma_fl(fl, carry):
            def fma(k, carry):
                fv_vec = plsc.load_gather(
                    fv_blk, [bc + k * 16 + lanes,
                             jnp.full((16,), t * 8 + fl, jnp.int32)])
                s = pl.ds(bc + k * 16, 16)
                acc_v[s] = (acc_v[s]
                            + bias_v[pl.ds(fl * CB + k * 16, 16)] * fv_vec)
                return carry

            lax.fori_loop(0, CB // 16, fma, 0)
            return carry

        lax.fori_loop(0, jnp.where(t == NT - 1, F - 8 * (NT - 1), 8),
                      fma_fl, 0)
        return carry

    lax.fori_loop(0, NCT, chunk_body, 0)
    pltpu.sync_copy(acc_v, first_out.at[pl.ds(b0, RW)])


_sc_prep = functools.partial(
    pl.kernel,
    out_type=[
        jax.ShapeDtypeStruct((NW * IDXW,), jnp.int32),
        jax.ShapeDtypeStruct((B,), jnp.float32),
    ],
    mesh=plsc.VectorSubcoreMesh(core_axis_name="c", subcore_axis_name="s"),
    scratch_types=[
        pltpu.VMEM((RW, FPAD), jnp.float32),
        pltpu.VMEM((RW, FPAD), jnp.float32),
        pltpu.VMEM((GPC,), jnp.int32),
        pltpu.VMEM((GPC,), jnp.float32),
        pltpu.VMEM((RW,), jnp.float32),
        pltpu.SemaphoreType.DMA,
    ],
    compiler_params=pltpu.CompilerParams(use_tc_tiling_on_sc=False,
                                         needs_layout_passes=False),
)(_sc_prep_body)


def _sc_gather_body(emb_hbm, idx_hbm, rows_out,
                    idx_a, idx_b, stage_a, stage_b, sem_a, sem_b):
    wid = lax.axis_index("s") * NC + lax.axis_index("c")
    b0 = wid * RW

    def start(ct, idx_v, stage_v, sem):
        pltpu.sync_copy(idx_hbm.at[pl.ds(wid * IDXW + ct * GPC, GPC)], idx_v)
        return pltpu.async_copy(emb_hbm.at[idx_v], stage_v, sem)

    def drain(ct, stage_v):
        c = ct // NT
        t = ct % NT
        bc = c * CB

        def wr(fl, carry):
            pltpu.sync_copy(
                stage_v.at[pl.ds(fl * CB, CB), :],
                rows_out.at[t, pl.ds(b0 + bc, CB), pl.ds(fl * D, D)])
            return carry

        lax.fori_loop(0, 8, wr, 0)

    start(0, idx_a, stage_a, sem_a).wait()

    def step(i, carry):
        ct = 2 * i
        cp = start(ct + 1, idx_b, stage_b, sem_b)
        drain(ct, stage_a)
        cp.wait()
        cp = start(ct + 2, idx_a, stage_a, sem_a)
        drain(ct + 1, stage_b)
        cp.wait()
        return carry

    lax.fori_loop(0, (NCT - 2) // 2, step, 0)
    cp = start(NCT - 1, idx_b, stage_b, sem_b)
    drain(NCT - 2, stage_a)
    cp.wait()
    drain(NCT - 1, stage_b)


_sc_gather = functools.partial(
    pl.kernel,
    out_type=jax.ShapeDtypeStruct((NT, B, 128), jnp.float32),
    mesh=plsc.VectorSubcoreMesh(core_axis_name="c", subcore_axis_name="s"),
    scratch_types=[
        pltpu.VMEM((GPC,), jnp.int32),
        pltpu.VMEM((GPC,), jnp.int32),
        pltpu.VMEM((GPC, D), jnp.float32),
        pltpu.VMEM((GPC, D), jnp.float32),
        pltpu.SemaphoreType.DMA,
        pltpu.SemaphoreType.DMA,
    ],
    compiler_params=pltpu.CompilerParams(use_tc_tiling_on_sc=False,
                                         needs_layout_passes=False),
)(_sc_gather_body)


BM = 256


def _tc_body(scal_ref, rows_ref, fv_ref, e_ref, p_ref,
             w1t_ref, b1_ref, w2t_ref, b2_ref, wph_ref, out_ref):
    fv = fv_ref[...]
    acc_s1 = jnp.zeros((BM, D), jnp.float32)
    acc_sq = jnp.zeros((BM, D), jnp.float32)
    acc_h = jnp.zeros((BM, 256), jnp.float32)
    for t in range(NT):
        ts = pl.ds(t * 128, 128)
        xt = rows_ref[t]
        fvet = jnp.dot(fv, e_ref[:, ts],
                       preferred_element_type=jnp.float32)
        fet = xt * fvet
        acc_s1 = acc_s1 + jnp.dot(fet, p_ref[ts, :],
                                  preferred_element_type=jnp.float32)
        acc_sq = acc_sq + jnp.dot(fet * fet, p_ref[ts, :],
                                  preferred_element_type=jnp.float32)
        acc_h = acc_h + jnp.dot(fet, w1t_ref[ts, :],
                                preferred_element_type=jnp.float32)
    second = 0.5 * jnp.sum(acc_s1 * acc_s1 - acc_sq, axis=1, keepdims=True)
    h = jnp.maximum(acc_h + b1_ref[...], 0.0)
    h = jnp.dot(h, w2t_ref[...], preferred_element_type=jnp.float32)
    h = jnp.maximum(h + b2_ref[...], 0.0)
    o = jnp.dot(h, wph_ref[...], preferred_element_type=jnp.float32)
    out_ref[...] = o + second * scal_ref[1] + scal_ref[2]


def _tc_fused(rows3, fv, e_mat, p_mat, w1t, b1r, w2t, b2r, wph, scal):
    grid = (B // BM,)
    full2 = lambda shape: pl.BlockSpec(shape, lambda i: (0, 0))
    return pl.pallas_call(
        _tc_body,
        grid=grid,
        in_specs=[
            pl.BlockSpec(memory_space=pltpu.SMEM),
            pl.BlockSpec((NT, BM, 128), lambda i: (0, i, 0)),
            pl.BlockSpec((BM, F), lambda i: (i, 0)),
            full2((F, FP)),
            full2((FP, D)),
            full2((FP, 256)),
            full2((1, 256)),
            full2((256, 128)),
            full2((1, 128)),
            full2((128, 1)),
        ],
        out_specs=pl.BlockSpec((BM, 1), lambda i: (i, 0)),
        out_shape=jax.ShapeDtypeStruct((B, 1), jnp.float32),
    )(scal, rows3, fv, e_mat, p_mat, w1t, b1r, w2t, b2r, wph)


def kernel(features, feature_values, emb_table, bias_table,
           W1, b1, W2, b2, Wp, bp):
    bias_flat = bias_table.reshape(-1)
    feat_p = jnp.pad(
        lax.bitcast_convert_type(features.astype(jnp.int32), jnp.float32),
        ((0, 0), (0, FPAD - F)))
    fv_p = jnp.pad(feature_values, ((0, 0), (0, FPAD - F)))

    idx_all, first = _sc_prep(bias_flat, feat_p, fv_p)
    rows3 = _sc_gather(emb_table, idx_all)

    cols = jnp.arange(FP, dtype=jnp.int32)
    fcol = (cols // 128) * 8 + (cols % 128) // D
    dcol = cols % D
    valid = fcol < F
    e_mat = ((fcol[None, :] == jnp.arange(F, dtype=jnp.int32)[:, None])
             & valid[None, :]).astype(jnp.float32)
    p_mat = ((dcol[:, None] == jnp.arange(D, dtype=jnp.int32)[None, :])
             & valid[:, None]).astype(jnp.float32)
    w1t_pad = jnp.pad(W1.T, ((0, FP - F * D), (0, 0)))
    scal = jnp.concatenate([Wp[0, :2], bp]).astype(jnp.float32)

    out = _tc_fused(rows3, feature_values, e_mat, p_mat,
                    w1t_pad, b1.reshape(1, -1), W2.T, b2.reshape(1, -1),
                    Wp[0, 2:].reshape(-1, 1), scal)
    return out.reshape(-1) + first * Wp[0, 0]
